# Initial kernel scaffold; baseline (speedup 1.0000x reference)
import jax
import jax.numpy as jnp
from jax.experimental import pallas as pl


def kernel(node_obs, adj, agent_id, params):
    raise NotImplementedError("write your pallas kernel here")


# speedup vs baseline: 155.3517x; 155.3517x over previous
import jax
import jax.numpy as jnp
import numpy as np
from jax.experimental import pallas as pl

B, N, A = 128, 64, 32
INPUT_DIM = 60
D = INPUT_DIM + 4
NUM_EMB, EMB_SIZE = 4, 16
EMBED_HIDDEN = 256
HIDDEN, HEADS = 128, 4
CONN, MAX_EDGE_DIST = 8, 0.5
HH = HEADS * HIDDEN
SQRT_H = np.sqrt(float(HIDDEN))
NEG = -1e30
LOWP = jax.lax.Precision.DEFAULT
HIGHP = jax.lax.Precision.HIGHEST
f32 = jnp.float32


def _ln(x, g, b):
    mu = jnp.mean(x, axis=-1, keepdims=True)
    xc = x - mu
    var = jnp.mean(xc * xc, axis=-1, keepdims=True)
    return xc / jnp.sqrt(var + 1e-5) * g + b


def _gnn_kernel(aid_ref, obs_ref, adj_ref,
                w1_ref, b1_ref, tab_ref,
                wh0_ref, bh0_ref, wh1_ref, bh1_ref,
                g1_ref, n1_ref, g2_ref, n2_ref, g3_ref, n3_ref,
                wq1_ref, bq1_ref, ve1_ref,
                wq2_ref, bq2_ref, ve2_ref,
                wq3_ref, bq3_ref, ve3_ref,
                out_ref):
    s = aid_ref[0, 0, 0]
    obs = obs_ref[0]
    adjb = adj_ref[0]

    iota_r = jax.lax.broadcasted_iota(jnp.int32, (A, 1), 0)
    iota_c = jax.lax.broadcasted_iota(jnp.int32, (1, A), 1)
    onehot_r = (iota_r == s).astype(f32)
    onehot_c = (iota_c == s).astype(f32)

    colmask = (jax.lax.broadcasted_iota(jnp.int32, (N, N), 1) != (N - 1))
    obs_f = jnp.where(colmask, obs, 0.0)
    et = obs[:, N - 1:N].astype(jnp.int32)
    emb = jnp.zeros((N, EMB_SIZE), f32)
    for t in range(NUM_EMB):
        emb = emb + (et == t).astype(f32) * tab_ref[t:t + 1, :]
    base = jnp.concatenate(
        [obs_f, emb, jnp.zeros((N, 128 - N - EMB_SIZE), f32)], axis=1)
    base_ag = base[:A]
    base_ga = base[A:]
    e_attr = (jax.lax.broadcasted_iota(jnp.int32, (1, 128), 1)
              == (N + EMB_SIZE)).astype(f32)

    aa_blk = adjb[:A, :A]
    row_s = jnp.sum(onehot_r * aa_blk, axis=0, keepdims=True)
    col_s = jnp.sum(aa_blk * onehot_c, axis=1, keepdims=True)
    big = jnp.float32(3.0e38)
    dist = jnp.where(iota_c == s, big, row_s)
    clf = jnp.zeros((1, A), f32)
    for _ in range(CONN):
        mn = jnp.min(dist)
        cand = dist == mn
        ii = jnp.min(jnp.where(cand, iota_c, A * 2))
        choose = iota_c == ii
        clf = jnp.where(choose, 1.0, clf)
        dist = jnp.where(choose, big, dist)
    clf_col = clf.reshape(A, 1)

    ag = adjb[:A, A:]
    ga = adjb[A:, :A]
    m_ag = (ag <= MAX_EDGE_DIST).astype(f32)
    m_ga = (ga <= MAX_EDGE_DIST).astype(f32)
    base_aid = jnp.sum(onehot_r * base_ag, axis=0, keepdims=True)

    ea3 = e_attr.reshape(1, 1, 128)
    z_ag = (base_ag[:, None, :] + ag[:, :, None] * ea3).reshape(A * A, 128)
    z_ga = (base_ga[:, None, :] + ga[:, :, None] * ea3).reshape(A * A, 128)
    z_out = base_aid + row_s.reshape(A, 1) * e_attr
    z_in = base_ag + col_s * e_attr
    z = jnp.concatenate([z_ag, z_ga, z_out, z_in], axis=0)

    h = jnp.maximum(jnp.dot(z, w1_ref[:], preferred_element_type=f32,
                            precision=LOWP) + b1_ref[:], 0.0)
    h = _ln(h, g1_ref[:], n1_ref[:])
    h = jnp.maximum(jnp.dot(h, wh0_ref[:], preferred_element_type=f32,
                            precision=LOWP) + bh0_ref[:], 0.0)
    h = _ln(h, g2_ref[:], n2_ref[:])
    h = jnp.maximum(jnp.dot(h, wh1_ref[:], preferred_element_type=f32,
                            precision=LOWP) + bh1_ref[:], 0.0)
    h = _ln(h, g3_ref[:], n3_ref[:])

    h_ag = h[:A * A].reshape(A, A, EMBED_HIDDEN)
    h_ga = h[A * A:2 * A * A].reshape(A, A, EMBED_HIDDEN)
    h_out = h[2 * A * A:2 * A * A + A]
    h_in = h[2 * A * A + A:]

    x1_ent = jnp.sum(h_ag * m_ag[:, :, None], axis=0)
    x1_agn = jnp.sum(h_ga * m_ga[:, :, None], axis=0)
    x1_agn = x1_agn + clf_col * h_out
    extra = jnp.sum(clf_col * h_in, axis=0, keepdims=True)
    x1_agn = x1_agn + onehot_r * extra
    x = jnp.concatenate([x1_agn, x1_ent], axis=0)

    m_aa = clf_col * onehot_c + onehot_r * clf
    mask_top = jnp.concatenate([m_aa, m_ga.T], axis=1)
    mask_bot = jnp.concatenate([m_ag.T, jnp.zeros((A, A), f32)], axis=1)
    maskf = jnp.concatenate([mask_top, mask_bot], axis=0)
    adj_t = adjb.T
    adj_tb = adj_t.astype(jnp.bfloat16).astype(f32)

    def tconv(xin, wq_ref, bq_ref, ve_ref):
        qkvs = jnp.dot(xin, wq_ref[:], preferred_element_type=f32,
                       precision=LOWP) + bq_ref[:]
        outs = []
        for hd in range(HEADS):
            lo = hd * HIDDEN
            qh = qkvs[:, lo:lo + HIDDEN]
            kh = qkvs[:, HH + lo:HH + lo + HIDDEN]
            vh = qkvs[:, 2 * HH + lo:2 * HH + lo + HIDDEN]
            weh = ve_ref[:, lo:lo + HIDDEN]
            qk = jax.lax.dot_general(qh, kh, (((1,), (1,)), ((), ())),
                                     preferred_element_type=f32,
                                     precision=HIGHP)
            qe = jnp.sum(qh * weh, axis=1, keepdims=True)
            sc = (qk + adj_tb * qe) / SQRT_H
            sc = jnp.where(maskf > 0.0, sc, NEG)
            mx = jnp.max(sc, axis=1, keepdims=True)
            ex = jnp.where(maskf > 0.0, jnp.exp(sc - mx), 0.0)
            ssum = jnp.sum(ex, axis=1, keepdims=True)
            alpha = ex / (ssum + 1e-16)
            oh = jnp.dot(alpha, vh, preferred_element_type=f32,
                         precision=HIGHP)
            coef = jnp.sum(alpha * adj_tb, axis=1, keepdims=True)
            outs.append(oh + coef * weh)
        att = jnp.concatenate(outs, axis=1)
        return jnp.maximum(att + qkvs[:, 3 * HH:], 0.0)

    x = tconv(x, wq1_ref, bq1_ref, ve1_ref)
    x = tconv(x, wq2_ref, bq2_ref, ve2_ref)
    x = tconv(x, wq3_ref, bq3_ref, ve3_ref)

    onehot_n = (jax.lax.broadcasted_iota(jnp.int32, (N, 1), 0) == s).astype(f32)
    out_ref[0] = jnp.sum(onehot_n * x, axis=0, keepdims=True)


def _full(shape):
    return pl.BlockSpec(shape, lambda b: (0,) * len(shape))


@jax.jit
def kernel(node_obs, adj, agent_id, params):
    W1 = params["lin1"]["W"]
    w1 = jnp.zeros((128, EMBED_HIDDEN), f32)
    w1 = w1.at[:D - 1].set(W1[:D - 1])
    w1 = w1.at[N:N + EMB_SIZE].set(W1[D - 1:D - 1 + EMB_SIZE])
    w1 = w1.at[N + EMB_SIZE].set(W1[D - 1 + EMB_SIZE])
    b1 = params["lin1"]["b"][None, :]
    tab = jnp.zeros((8, EMB_SIZE), f32).at[:NUM_EMB].set(params["entity_embed"])

    wh0, bh0 = params["lin_h"][0]["W"], params["lin_h"][0]["b"][None, :]
    wh1, bh1 = params["lin_h"][1]["W"], params["lin_h"][1]["b"][None, :]
    g1, n1 = params["ln1"]["g"][None, :], params["ln1"]["b"][None, :]
    g2, n2 = params["ln_h"][0]["g"][None, :], params["ln_h"][0]["b"][None, :]
    g3, n3 = params["ln_h"][1]["g"][None, :], params["ln_h"][1]["b"][None, :]

    def pack(p):
        wq = jnp.concatenate([p["q"]["W"], p["k"]["W"], p["v"]["W"],
                              p["skip"]["W"]], axis=1)
        bq = jnp.concatenate([p["q"]["b"], p["k"]["b"], p["v"]["b"],
                              p["skip"]["b"]])[None, :]
        ve = p["e"]["W"].reshape(1, HH).astype(jnp.bfloat16).astype(f32)
        return wq, bq, ve

    wq1, bq1, ve1 = pack(params["gnn1"])
    wq2, bq2, ve2 = pack(params["gnn2"][0])
    wq3, bq3, ve3 = pack(params["gnn2"][1])

    aid3 = agent_id.astype(jnp.int32).reshape(B, 1, 1)

    out = pl.pallas_call(
        _gnn_kernel,
        grid=(B,),
        in_specs=[
            pl.BlockSpec((1, 1, 1), lambda b: (b, 0, 0)),
            pl.BlockSpec((1, N, N), lambda b: (b, 0, 0)),
            pl.BlockSpec((1, N, N), lambda b: (b, 0, 0)),
            _full(w1.shape), _full(b1.shape), _full(tab.shape),
            _full(wh0.shape), _full(bh0.shape),
            _full(wh1.shape), _full(bh1.shape),
            _full(g1.shape), _full(n1.shape), _full(g2.shape),
            _full(n2.shape), _full(g3.shape), _full(n3.shape),
            _full(wq1.shape), _full(bq1.shape), _full(ve1.shape),
            _full(wq2.shape), _full(bq2.shape), _full(ve2.shape),
            _full(wq3.shape), _full(bq3.shape), _full(ve3.shape),
        ],
        out_specs=pl.BlockSpec((1, 1, HH), lambda b: (b, 0, 0)),
        out_shape=jax.ShapeDtypeStruct((B, 1, HH), f32),
    )(aid3, node_obs, adj,
      w1, b1, tab, wh0, bh0, wh1, bh1,
      g1, n1, g2, n2, g3, n3,
      wq1, bq1, ve1, wq2, bq2, ve2, wq3, bq3, ve3)
    return out.reshape(B, HH)


# speedup vs baseline: 211.8651x; 1.3638x over previous
import jax
import jax.numpy as jnp
import numpy as np
from jax.experimental import pallas as pl

B, N, A = 128, 64, 32
INPUT_DIM = 60
D = INPUT_DIM + 4
NUM_EMB, EMB_SIZE = 4, 16
EMBED_HIDDEN = 256
HIDDEN, HEADS = 128, 4
CONN, MAX_EDGE_DIST = 8, 0.5
HH = HEADS * HIDDEN
SQRT_H = np.sqrt(float(HIDDEN))
NEG = -1e30
LOWP = jax.lax.Precision.DEFAULT
HIGHP = jax.lax.Precision.HIGHEST
f32 = jnp.float32
GB = 2
NE = 2 * A * A + 2 * A


def _ln(x, g, b):
    mu = jnp.mean(x, axis=-1, keepdims=True)
    xc = x - mu
    var = jnp.mean(xc * xc, axis=-1, keepdims=True)
    return xc / jnp.sqrt(var + 1e-5) * g + b


def _gnn_kernel(aid_ref, obs_ref, adj_ref,
                w1_ref, b1_ref, tab_ref,
                wh0_ref, bh0_ref, wh1_ref, bh1_ref,
                g1_ref, n1_ref, g2_ref, n2_ref, g3_ref, n3_ref,
                wq1_ref, bq1_ref, ve1_ref,
                wq2_ref, bq2_ref, ve2_ref,
                wkv3_ref, bkv3_ref, wqs3_ref, bqs3_ref, ve3_ref,
                out_ref):
    iota_r = jax.lax.broadcasted_iota(jnp.int32, (A, 1), 0)
    iota_c = jax.lax.broadcasted_iota(jnp.int32, (1, A), 1)
    e_attr = (jax.lax.broadcasted_iota(jnp.int32, (1, 128), 1)
              == (N + EMB_SIZE)).astype(f32)
    colmask = (jax.lax.broadcasted_iota(jnp.int32, (N, N), 1) != (N - 1))
    iota_n = jax.lax.broadcasted_iota(jnp.int32, (N, 1), 0)

    zs, gdata = [], []
    for g in range(GB):
        s = aid_ref[g, 0, 0]
        obs = obs_ref[g]
        adjb = adj_ref[g]
        onehot_r = (iota_r == s).astype(f32)
        onehot_c = (iota_c == s).astype(f32)
        onehot_n = (iota_n == s).astype(f32)

        obs_f = jnp.where(colmask, obs, 0.0)
        et = obs[:, N - 1:N].astype(jnp.int32)
        emb = jnp.zeros((N, EMB_SIZE), f32)
        for t in range(NUM_EMB):
            emb = emb + (et == t).astype(f32) * tab_ref[t:t + 1, :]
        base = jnp.concatenate(
            [obs_f, emb, jnp.zeros((N, 128 - N - EMB_SIZE), f32)], axis=1)
        base_ag = base[:A]
        base_ga = base[A:]

        aa_blk = adjb[:A, :A]
        row_s = jnp.sum(onehot_r * aa_blk, axis=0, keepdims=True)
        col_s = jnp.sum(aa_blk * onehot_c, axis=1, keepdims=True)
        big = jnp.float32(3.0e38)
        dist = jnp.where(iota_c == s, big, row_s)
        clf = jnp.zeros((1, A), f32)
        for _ in range(CONN):
            mn = jnp.min(dist)
            cand = dist == mn
            ii = jnp.min(jnp.where(cand, iota_c, A * 2))
            choose = iota_c == ii
            clf = jnp.where(choose, 1.0, clf)
            dist = jnp.where(choose, big, dist)
        clf_col = clf.reshape(A, 1)

        ag = adjb[:A, A:]
        ga = adjb[A:, :A]
        m_ag = (ag <= MAX_EDGE_DIST).astype(f32)
        m_ga = (ga <= MAX_EDGE_DIST).astype(f32)
        base_aid = jnp.sum(onehot_r * base_ag, axis=0, keepdims=True)

        ea3 = e_attr.reshape(1, 1, 128)
        z_ag = (base_ag[:, None, :] + ag[:, :, None] * ea3).reshape(A * A, 128)
        z_ga = (base_ga[:, None, :] + ga[:, :, None] * ea3).reshape(A * A, 128)
        z_out = base_aid + row_s.reshape(A, 1) * e_attr
        z_in = base_ag + col_s * e_attr
        zs.append(jnp.concatenate([z_ag, z_ga, z_out, z_in], axis=0))

        m_aa = clf_col * onehot_c + onehot_r * clf
        mask_top = jnp.concatenate([m_aa, m_ga.T], axis=1)
        mask_bot = jnp.concatenate([m_ag.T, jnp.zeros((A, A), f32)], axis=1)
        maskf = jnp.concatenate([mask_top, mask_bot], axis=0)
        adj_t = adjb.T
        adj_tb = adj_t.astype(jnp.bfloat16).astype(f32)
        gdata.append((onehot_r, onehot_n, clf_col, m_ag, m_ga, maskf, adj_tb))

    z = jnp.concatenate(zs, axis=0)
    h = jnp.maximum(jnp.dot(z, w1_ref[:], preferred_element_type=f32,
                            precision=LOWP) + b1_ref[:], 0.0)
    h = _ln(h, g1_ref[:], n1_ref[:])
    h = jnp.maximum(jnp.dot(h, wh0_ref[:], preferred_element_type=f32,
                            precision=LOWP) + bh0_ref[:], 0.0)
    h = _ln(h, g2_ref[:], n2_ref[:])
    h = jnp.maximum(jnp.dot(h, wh1_ref[:], preferred_element_type=f32,
                            precision=LOWP) + bh1_ref[:], 0.0)
    h = _ln(h, g3_ref[:], n3_ref[:])

    xs = []
    for g in range(GB):
        onehot_r, _, clf_col, m_ag, m_ga, _, _ = gdata[g]
        hg = h[g * NE:(g + 1) * NE]
        h_ag = hg[:A * A].reshape(A, A, EMBED_HIDDEN)
        h_ga = hg[A * A:2 * A * A].reshape(A, A, EMBED_HIDDEN)
        h_out = hg[2 * A * A:2 * A * A + A]
        h_in = hg[2 * A * A + A:]
        x1_ent = jnp.sum(h_ag * m_ag[:, :, None], axis=0)
        x1_agn = jnp.sum(h_ga * m_ga[:, :, None], axis=0)
        x1_agn = x1_agn + clf_col * h_out
        extra = jnp.sum(clf_col * h_in, axis=0, keepdims=True)
        x1_agn = x1_agn + onehot_r * extra
        xs.append(jnp.concatenate([x1_agn, x1_ent], axis=0))
    x = jnp.concatenate(xs, axis=0)

    mask4 = [jnp.concatenate([gd[5]] * HEADS, axis=1) for gd in gdata]

    def attend(qkvs_g, maskf, adj_tb, ve_ref):
        scs = []
        for hd in range(HEADS):
            lo = hd * HIDDEN
            qh = qkvs_g[:, lo:lo + HIDDEN]
            kh = qkvs_g[:, HH + lo:HH + lo + HIDDEN]
            weh = ve_ref[:, lo:lo + HIDDEN]
            qk = jax.lax.dot_general(qh, kh, (((1,), (1,)), ((), ())),
                                     preferred_element_type=f32,
                                     precision=HIGHP)
            qe = jnp.sum(qh * weh, axis=1, keepdims=True)
            scs.append((qk + adj_tb * qe) / SQRT_H)
        sc = jnp.concatenate(scs, axis=1)
        m4 = jnp.concatenate([maskf] * HEADS, axis=1)
        sc = jnp.where(m4 > 0.0, sc, NEG)
        scr = sc.reshape(N, HEADS, N)
        mx = jnp.max(scr, axis=2, keepdims=True)
        ex = jnp.where(m4.reshape(N, HEADS, N) > 0.0,
                       jnp.exp(scr - mx), 0.0)
        ssum = jnp.sum(ex, axis=2, keepdims=True)
        alpha = (ex / (ssum + 1e-16)).reshape(N, HEADS * N)
        outs = []
        for hd in range(HEADS):
            lo = hd * HIDDEN
            vh = qkvs_g[:, 2 * HH + lo:2 * HH + lo + HIDDEN]
            weh = ve_ref[:, lo:lo + HIDDEN]
            al = alpha[:, hd * N:(hd + 1) * N]
            oh = jnp.dot(al, vh, preferred_element_type=f32, precision=HIGHP)
            coef = jnp.sum(al * adj_tb, axis=1, keepdims=True)
            outs.append(oh + coef * weh)
        return jnp.concatenate(outs, axis=1)

    def tconv_full(xin, wq_ref, bq_ref, ve_ref):
        qkvs = jnp.dot(xin, wq_ref[:], preferred_element_type=f32,
                       precision=LOWP) + bq_ref[:]
        rows = []
        for g in range(GB):
            qg = qkvs[g * N:(g + 1) * N]
            att = attend(qg, gdata[g][5], gdata[g][6], ve_ref)
            rows.append(jnp.maximum(att + qg[:, 3 * HH:], 0.0))
        return jnp.concatenate(rows, axis=0)

    x = tconv_full(x, wq1_ref, bq1_ref, ve1_ref)
    x = tconv_full(x, wq2_ref, bq2_ref, ve2_ref)

    kv3 = jnp.dot(x, wkv3_ref[:], preferred_element_type=f32,
                  precision=LOWP) + bkv3_ref[:]
    xrows = jnp.concatenate(
        [jnp.sum(gdata[g][1] * x[g * N:(g + 1) * N], axis=0, keepdims=True)
         for g in range(GB)], axis=0)
    qs3 = jnp.dot(xrows, wqs3_ref[:], preferred_element_type=f32,
                  precision=LOWP) + bqs3_ref[:]
    for g in range(GB):
        onehot_n = gdata[g][1]
        maskf, adj_tb = gdata[g][5], gdata[g][6]
        mrow = jnp.sum(onehot_n * maskf, axis=0, keepdims=True)
        arow = jnp.sum(onehot_n * adj_tb, axis=0, keepdims=True)
        kvg = kv3[g * N:(g + 1) * N]
        scs = []
        for hd in range(HEADS):
            lo = hd * HIDDEN
            qh = qs3[g:g + 1, lo:lo + HIDDEN]
            kh = kvg[:, lo:lo + HIDDEN]
            weh = ve3_ref[:, lo:lo + HIDDEN]
            qk = jax.lax.dot_general(qh, kh, (((1,), (1,)), ((), ())),
                                     preferred_element_type=f32,
                                     precision=HIGHP)
            qe = jnp.sum(qh * weh, axis=1, keepdims=True)
            scs.append((qk + arow * qe) / SQRT_H)
        sc = jnp.concatenate(scs, axis=1)
        m4 = jnp.concatenate([mrow] * HEADS, axis=1)
        sc = jnp.where(m4 > 0.0, sc, NEG)
        scr = sc.reshape(1, HEADS, N)
        mx = jnp.max(scr, axis=2, keepdims=True)
        ex = jnp.where(m4.reshape(1, HEADS, N) > 0.0, jnp.exp(scr - mx), 0.0)
        ssum = jnp.sum(ex, axis=2, keepdims=True)
        alpha = (ex / (ssum + 1e-16)).reshape(1, HEADS * N)
        outs = []
        for hd in range(HEADS):
            lo = hd * HIDDEN
            vh = kvg[:, HH + lo:HH + lo + HIDDEN]
            weh = ve3_ref[:, lo:lo + HIDDEN]
            al = alpha[:, hd * N:(hd + 1) * N]
            oh = jnp.dot(al, vh, preferred_element_type=f32, precision=HIGHP)
            coef = jnp.sum(al * arow, axis=1, keepdims=True)
            outs.append(oh + coef * weh)
        att = jnp.concatenate(outs, axis=1)
        out_ref[g] = jnp.maximum(att + qs3[g:g + 1, HH:], 0.0)


def _full(shape):
    return pl.BlockSpec(shape, lambda b: (0,) * len(shape))


@jax.jit
def kernel(node_obs, adj, agent_id, params):
    W1 = params["lin1"]["W"]
    w1 = jnp.zeros((128, EMBED_HIDDEN), f32)
    w1 = w1.at[:D - 1].set(W1[:D - 1])
    w1 = w1.at[N:N + EMB_SIZE].set(W1[D - 1:D - 1 + EMB_SIZE])
    w1 = w1.at[N + EMB_SIZE].set(W1[D - 1 + EMB_SIZE])
    b1 = params["lin1"]["b"][None, :]
    tab = jnp.zeros((8, EMB_SIZE), f32).at[:NUM_EMB].set(params["entity_embed"])

    wh0, bh0 = params["lin_h"][0]["W"], params["lin_h"][0]["b"][None, :]
    wh1, bh1 = params["lin_h"][1]["W"], params["lin_h"][1]["b"][None, :]
    g1, n1 = params["ln1"]["g"][None, :], params["ln1"]["b"][None, :]
    g2, n2 = params["ln_h"][0]["g"][None, :], params["ln_h"][0]["b"][None, :]
    g3, n3 = params["ln_h"][1]["g"][None, :], params["ln_h"][1]["b"][None, :]

    def pack(p):
        wq = jnp.concatenate([p["q"]["W"], p["k"]["W"], p["v"]["W"],
                              p["skip"]["W"]], axis=1)
        bq = jnp.concatenate([p["q"]["b"], p["k"]["b"], p["v"]["b"],
                              p["skip"]["b"]])[None, :]
        ve = p["e"]["W"].reshape(1, HH).astype(jnp.bfloat16).astype(f32)
        return wq, bq, ve

    wq1, bq1, ve1 = pack(params["gnn1"])
    wq2, bq2, ve2 = pack(params["gnn2"][0])
    p3 = params["gnn2"][1]
    wkv3 = jnp.concatenate([p3["k"]["W"], p3["v"]["W"]], axis=1)
    bkv3 = jnp.concatenate([p3["k"]["b"], p3["v"]["b"]])[None, :]
    wqs3 = jnp.concatenate([p3["q"]["W"], p3["skip"]["W"]], axis=1)
    bqs3 = jnp.concatenate([p3["q"]["b"], p3["skip"]["b"]])[None, :]
    ve3 = p3["e"]["W"].reshape(1, HH).astype(jnp.bfloat16).astype(f32)

    aid3 = agent_id.astype(jnp.int32).reshape(B, 1, 1)

    out = pl.pallas_call(
        _gnn_kernel,
        grid=(B // GB,),
        in_specs=[
            pl.BlockSpec((GB, 1, 1), lambda b: (b, 0, 0)),
            pl.BlockSpec((GB, N, N), lambda b: (b, 0, 0)),
            pl.BlockSpec((GB, N, N), lambda b: (b, 0, 0)),
            _full(w1.shape), _full(b1.shape), _full(tab.shape),
            _full(wh0.shape), _full(bh0.shape),
            _full(wh1.shape), _full(bh1.shape),
            _full(g1.shape), _full(n1.shape), _full(g2.shape),
            _full(n2.shape), _full(g3.shape), _full(n3.shape),
            _full(wq1.shape), _full(bq1.shape), _full(ve1.shape),
            _full(wq2.shape), _full(bq2.shape), _full(ve2.shape),
            _full(wkv3.shape), _full(bkv3.shape),
            _full(wqs3.shape), _full(bqs3.shape), _full(ve3.shape),
        ],
        out_specs=pl.BlockSpec((GB, 1, HH), lambda b: (b, 0, 0)),
        out_shape=jax.ShapeDtypeStruct((B, 1, HH), f32),
    )(aid3, node_obs, adj,
      w1, b1, tab, wh0, bh0, wh1, bh1,
      g1, n1, g2, n2, g3, n3,
      wq1, bq1, ve1, wq2, bq2, ve2,
      wkv3, bkv3, wqs3, bqs3, ve3)
    return out.reshape(B, HH)


# speedup vs baseline: 224.8968x; 1.0615x over previous
import jax
import jax.numpy as jnp
import numpy as np
from jax.experimental import pallas as pl

B, N, A = 128, 64, 32
INPUT_DIM = 60
D = INPUT_DIM + 4
NUM_EMB, EMB_SIZE = 4, 16
EMBED_HIDDEN = 256
HIDDEN, HEADS = 128, 4
CONN, MAX_EDGE_DIST = 8, 0.5
HH = HEADS * HIDDEN
SQRT_H = np.sqrt(float(HIDDEN))
NEG = -1e30
LOWP = jax.lax.Precision.DEFAULT
HIGHP = jax.lax.Precision.HIGHEST
f32 = jnp.float32
GB = 4
NE = 2 * A * A + 2 * A


def _ln(x, g, b):
    mu = jnp.mean(x, axis=-1, keepdims=True)
    xc = x - mu
    var = jnp.mean(xc * xc, axis=-1, keepdims=True)
    return xc / jnp.sqrt(var + 1e-5) * g + b


def _gnn_kernel(aid_ref, obs_ref, adj_ref,
                w1_ref, b1_ref, tab_ref,
                wh0_ref, bh0_ref, wh1_ref, bh1_ref,
                g1_ref, n1_ref, g2_ref, n2_ref, g3_ref, n3_ref,
                wq1_ref, bq1_ref, ve1_ref,
                wq2_ref, bq2_ref, ve2_ref,
                wkv3_ref, bkv3_ref, wqs3_ref, bqs3_ref, ve3_ref,
                out_ref):
    iota_r = jax.lax.broadcasted_iota(jnp.int32, (A, 1), 0)
    iota_c = jax.lax.broadcasted_iota(jnp.int32, (1, A), 1)
    e_attr = (jax.lax.broadcasted_iota(jnp.int32, (1, 128), 1)
              == (N + EMB_SIZE)).astype(f32)
    colmask = (jax.lax.broadcasted_iota(jnp.int32, (N, N), 1) != (N - 1))
    iota_n = jax.lax.broadcasted_iota(jnp.int32, (N, 1), 0)

    zs, gdata = [], []
    for g in range(GB):
        s = aid_ref[g, 0, 0]
        obs = obs_ref[g]
        adjb = adj_ref[g]
        onehot_r = (iota_r == s).astype(f32)
        onehot_c = (iota_c == s).astype(f32)
        onehot_n = (iota_n == s).astype(f32)

        obs_f = jnp.where(colmask, obs, 0.0)
        et = obs[:, N - 1:N].astype(jnp.int32)
        emb = jnp.zeros((N, EMB_SIZE), f32)
        for t in range(NUM_EMB):
            emb = emb + (et == t).astype(f32) * tab_ref[t:t + 1, :]
        base = jnp.concatenate(
            [obs_f, emb, jnp.zeros((N, 128 - N - EMB_SIZE), f32)], axis=1)
        base_ag = base[:A]
        base_ga = base[A:]

        aa_blk = adjb[:A, :A]
        row_s = jnp.sum(onehot_r * aa_blk, axis=0, keepdims=True)
        col_s = jnp.sum(aa_blk * onehot_c, axis=1, keepdims=True)
        big = jnp.float32(3.0e38)
        dist = jnp.where(iota_c == s, big, row_s)
        clf = jnp.zeros((1, A), f32)
        for _ in range(CONN):
            mn = jnp.min(dist)
            cand = dist == mn
            ii = jnp.min(jnp.where(cand, iota_c, A * 2))
            choose = iota_c == ii
            clf = jnp.where(choose, 1.0, clf)
            dist = jnp.where(choose, big, dist)
        clf_col = clf.reshape(A, 1)

        ag = adjb[:A, A:]
        ga = adjb[A:, :A]
        m_ag = (ag <= MAX_EDGE_DIST).astype(f32)
        m_ga = (ga <= MAX_EDGE_DIST).astype(f32)
        base_aid = jnp.sum(onehot_r * base_ag, axis=0, keepdims=True)

        ea3 = e_attr.reshape(1, 1, 128)
        z_ag = (base_ag[:, None, :] + ag[:, :, None] * ea3).reshape(A * A, 128)
        z_ga = (base_ga[:, None, :] + ga[:, :, None] * ea3).reshape(A * A, 128)
        z_out = base_aid + row_s.reshape(A, 1) * e_attr
        z_in = base_ag + col_s * e_attr
        zs.append(jnp.concatenate([z_ag, z_ga, z_out, z_in], axis=0))

        m_aa = clf_col * onehot_c + onehot_r * clf
        mask_top = jnp.concatenate([m_aa, m_ga.T], axis=1)
        mask_bot = jnp.concatenate([m_ag.T, jnp.zeros((A, A), f32)], axis=1)
        maskf = jnp.concatenate([mask_top, mask_bot], axis=0)
        adj_t = adjb.T
        adj_tb = adj_t.astype(jnp.bfloat16).astype(f32)
        gdata.append((onehot_r, onehot_n, clf_col, m_ag, m_ga, maskf, adj_tb))

    z = jnp.concatenate(zs, axis=0)
    h = jnp.maximum(jnp.dot(z, w1_ref[:], preferred_element_type=f32,
                            precision=LOWP) + b1_ref[:], 0.0)
    h = _ln(h, g1_ref[:], n1_ref[:])
    h = jnp.maximum(jnp.dot(h, wh0_ref[:], preferred_element_type=f32,
                            precision=LOWP) + bh0_ref[:], 0.0)
    h = _ln(h, g2_ref[:], n2_ref[:])
    h = jnp.maximum(jnp.dot(h, wh1_ref[:], preferred_element_type=f32,
                            precision=LOWP) + bh1_ref[:], 0.0)
    h = _ln(h, g3_ref[:], n3_ref[:])

    xs = []
    for g in range(GB):
        onehot_r, _, clf_col, m_ag, m_ga, _, _ = gdata[g]
        hg = h[g * NE:(g + 1) * NE]
        h_ag = hg[:A * A].reshape(A, A, EMBED_HIDDEN)
        h_ga = hg[A * A:2 * A * A].reshape(A, A, EMBED_HIDDEN)
        h_out = hg[2 * A * A:2 * A * A + A]
        h_in = hg[2 * A * A + A:]
        x1_ent = jnp.sum(h_ag * m_ag[:, :, None], axis=0)
        x1_agn = jnp.sum(h_ga * m_ga[:, :, None], axis=0)
        x1_agn = x1_agn + clf_col * h_out
        extra = jnp.sum(clf_col * h_in, axis=0, keepdims=True)
        x1_agn = x1_agn + onehot_r * extra
        xs.append(jnp.concatenate([x1_agn, x1_ent], axis=0))
    x = jnp.concatenate(xs, axis=0)

    mask4 = [jnp.concatenate([gd[5]] * HEADS, axis=1) for gd in gdata]

    def attend(qkvs_g, maskf, adj_tb, ve_ref):
        scs = []
        for hd in range(HEADS):
            lo = hd * HIDDEN
            qh = qkvs_g[:, lo:lo + HIDDEN]
            kh = qkvs_g[:, HH + lo:HH + lo + HIDDEN]
            weh = ve_ref[:, lo:lo + HIDDEN]
            qk = jax.lax.dot_general(qh, kh, (((1,), (1,)), ((), ())),
                                     preferred_element_type=f32,
                                     precision=HIGHP)
            qe = jnp.sum(qh * weh, axis=1, keepdims=True)
            scs.append((qk + adj_tb * qe) / SQRT_H)
        sc = jnp.concatenate(scs, axis=1)
        m4 = jnp.concatenate([maskf] * HEADS, axis=1)
        sc = jnp.where(m4 > 0.0, sc, NEG)
        scr = sc.reshape(N, HEADS, N)
        mx = jnp.max(scr, axis=2, keepdims=True)
        ex = jnp.where(m4.reshape(N, HEADS, N) > 0.0,
                       jnp.exp(scr - mx), 0.0)
        ssum = jnp.sum(ex, axis=2, keepdims=True)
        alpha = (ex / (ssum + 1e-16)).reshape(N, HEADS * N)
        outs = []
        for hd in range(HEADS):
            lo = hd * HIDDEN
            vh = qkvs_g[:, 2 * HH + lo:2 * HH + lo + HIDDEN]
            weh = ve_ref[:, lo:lo + HIDDEN]
            al = alpha[:, hd * N:(hd + 1) * N]
            oh = jnp.dot(al, vh, preferred_element_type=f32, precision=HIGHP)
            coef = jnp.sum(al * adj_tb, axis=1, keepdims=True)
            outs.append(oh + coef * weh)
        return jnp.concatenate(outs, axis=1)

    def tconv_full(xin, wq_ref, bq_ref, ve_ref):
        qkvs = jnp.dot(xin, wq_ref[:], preferred_element_type=f32,
                       precision=LOWP) + bq_ref[:]
        rows = []
        for g in range(GB):
            qg = qkvs[g * N:(g + 1) * N]
            att = attend(qg, gdata[g][5], gdata[g][6], ve_ref)
            rows.append(jnp.maximum(att + qg[:, 3 * HH:], 0.0))
        return jnp.concatenate(rows, axis=0)

    x = tconv_full(x, wq1_ref, bq1_ref, ve1_ref)
    x = tconv_full(x, wq2_ref, bq2_ref, ve2_ref)

    kv3 = jnp.dot(x, wkv3_ref[:], preferred_element_type=f32,
                  precision=LOWP) + bkv3_ref[:]
    xrows = jnp.concatenate(
        [jnp.sum(gdata[g][1] * x[g * N:(g + 1) * N], axis=0, keepdims=True)
         for g in range(GB)], axis=0)
    qs3 = jnp.dot(xrows, wqs3_ref[:], preferred_element_type=f32,
                  precision=LOWP) + bqs3_ref[:]
    for g in range(GB):
        onehot_n = gdata[g][1]
        maskf, adj_tb = gdata[g][5], gdata[g][6]
        mrow = jnp.sum(onehot_n * maskf, axis=0, keepdims=True)
        arow = jnp.sum(onehot_n * adj_tb, axis=0, keepdims=True)
        kvg = kv3[g * N:(g + 1) * N]
        scs = []
        for hd in range(HEADS):
            lo = hd * HIDDEN
            qh = qs3[g:g + 1, lo:lo + HIDDEN]
            kh = kvg[:, lo:lo + HIDDEN]
            weh = ve3_ref[:, lo:lo + HIDDEN]
            qk = jax.lax.dot_general(qh, kh, (((1,), (1,)), ((), ())),
                                     preferred_element_type=f32,
                                     precision=HIGHP)
            qe = jnp.sum(qh * weh, axis=1, keepdims=True)
            scs.append((qk + arow * qe) / SQRT_H)
        sc = jnp.concatenate(scs, axis=1)
        m4 = jnp.concatenate([mrow] * HEADS, axis=1)
        sc = jnp.where(m4 > 0.0, sc, NEG)
        scr = sc.reshape(1, HEADS, N)
        mx = jnp.max(scr, axis=2, keepdims=True)
        ex = jnp.where(m4.reshape(1, HEADS, N) > 0.0, jnp.exp(scr - mx), 0.0)
        ssum = jnp.sum(ex, axis=2, keepdims=True)
        alpha = (ex / (ssum + 1e-16)).reshape(1, HEADS * N)
        outs = []
        for hd in range(HEADS):
            lo = hd * HIDDEN
            vh = kvg[:, HH + lo:HH + lo + HIDDEN]
            weh = ve3_ref[:, lo:lo + HIDDEN]
            al = alpha[:, hd * N:(hd + 1) * N]
            oh = jnp.dot(al, vh, preferred_element_type=f32, precision=HIGHP)
            coef = jnp.sum(al * arow, axis=1, keepdims=True)
            outs.append(oh + coef * weh)
        att = jnp.concatenate(outs, axis=1)
        out_ref[g] = jnp.maximum(att + qs3[g:g + 1, HH:], 0.0)


def _full(shape):
    return pl.BlockSpec(shape, lambda b: (0,) * len(shape))


@jax.jit
def kernel(node_obs, adj, agent_id, params):
    W1 = params["lin1"]["W"]
    w1 = jnp.zeros((128, EMBED_HIDDEN), f32)
    w1 = w1.at[:D - 1].set(W1[:D - 1])
    w1 = w1.at[N:N + EMB_SIZE].set(W1[D - 1:D - 1 + EMB_SIZE])
    w1 = w1.at[N + EMB_SIZE].set(W1[D - 1 + EMB_SIZE])
    b1 = params["lin1"]["b"][None, :]
    tab = jnp.zeros((8, EMB_SIZE), f32).at[:NUM_EMB].set(params["entity_embed"])

    wh0, bh0 = params["lin_h"][0]["W"], params["lin_h"][0]["b"][None, :]
    wh1, bh1 = params["lin_h"][1]["W"], params["lin_h"][1]["b"][None, :]
    g1, n1 = params["ln1"]["g"][None, :], params["ln1"]["b"][None, :]
    g2, n2 = params["ln_h"][0]["g"][None, :], params["ln_h"][0]["b"][None, :]
    g3, n3 = params["ln_h"][1]["g"][None, :], params["ln_h"][1]["b"][None, :]

    def pack(p):
        wq = jnp.concatenate([p["q"]["W"], p["k"]["W"], p["v"]["W"],
                              p["skip"]["W"]], axis=1)
        bq = jnp.concatenate([p["q"]["b"], p["k"]["b"], p["v"]["b"],
                              p["skip"]["b"]])[None, :]
        ve = p["e"]["W"].reshape(1, HH).astype(jnp.bfloat16).astype(f32)
        return wq, bq, ve

    wq1, bq1, ve1 = pack(params["gnn1"])
    wq2, bq2, ve2 = pack(params["gnn2"][0])
    p3 = params["gnn2"][1]
    wkv3 = jnp.concatenate([p3["k"]["W"], p3["v"]["W"]], axis=1)
    bkv3 = jnp.concatenate([p3["k"]["b"], p3["v"]["b"]])[None, :]
    wqs3 = jnp.concatenate([p3["q"]["W"], p3["skip"]["W"]], axis=1)
    bqs3 = jnp.concatenate([p3["q"]["b"], p3["skip"]["b"]])[None, :]
    ve3 = p3["e"]["W"].reshape(1, HH).astype(jnp.bfloat16).astype(f32)

    aid3 = agent_id.astype(jnp.int32).reshape(B, 1, 1)

    out = pl.pallas_call(
        _gnn_kernel,
        grid=(B // GB,),
        in_specs=[
            pl.BlockSpec((GB, 1, 1), lambda b: (b, 0, 0)),
            pl.BlockSpec((GB, N, N), lambda b: (b, 0, 0)),
            pl.BlockSpec((GB, N, N), lambda b: (b, 0, 0)),
            _full(w1.shape), _full(b1.shape), _full(tab.shape),
            _full(wh0.shape), _full(bh0.shape),
            _full(wh1.shape), _full(bh1.shape),
            _full(g1.shape), _full(n1.shape), _full(g2.shape),
            _full(n2.shape), _full(g3.shape), _full(n3.shape),
            _full(wq1.shape), _full(bq1.shape), _full(ve1.shape),
            _full(wq2.shape), _full(bq2.shape), _full(ve2.shape),
            _full(wkv3.shape), _full(bkv3.shape),
            _full(wqs3.shape), _full(bqs3.shape), _full(ve3.shape),
        ],
        out_specs=pl.BlockSpec((GB, 1, HH), lambda b: (b, 0, 0)),
        out_shape=jax.ShapeDtypeStruct((B, 1, HH), f32),
    )(aid3, node_obs, adj,
      w1, b1, tab, wh0, bh0, wh1, bh1,
      g1, n1, g2, n2, g3, n3,
      wq1, bq1, ve1, wq2, bq2, ve2,
      wkv3, bkv3, wqs3, bqs3, ve3)
    return out.reshape(B, HH)
